# Initial kernel scaffold; baseline (speedup 1.0000x reference)
import jax
import jax.numpy as jnp
from jax.experimental import pallas as pl


def kernel(hidden_states, W_emb, b_emb, Wq, bq, Wk, bk, Wv, bv, Wo, bo, ln_attn_g, ln_attn_b, Wr, W1, b1, W2, b2, ln_out_g, ln_out_b):
    raise NotImplementedError("write your pallas kernel here")


# speedup vs baseline: 1.1096x; 1.1096x over previous
import math

import jax
import jax.numpy as jnp
from jax.experimental import pallas as pl
from jax.experimental.pallas import tpu as pltpu

EMBD = 128
D = 1024
NH = 16
DH = 64
NE = 64
DFF = 2048
NLAYERS = 2
LNEPS = 1e-12
T = 2048
CAP = 64
QB = 256
SB = 512
TB = 256

_BF = jnp.bfloat16
_F32 = jnp.float32


def _embed_kernel(x_ref, w_ref, b_ref, o_ref):
    o_ref[...] = (
        jnp.dot(x_ref[...], w_ref[...], preferred_element_type=_F32) + b_ref[...]
    )


def _embed(x, w, b):
    return pl.pallas_call(
        _embed_kernel,
        out_shape=jax.ShapeDtypeStruct((T, D), _F32),
    )(x, w, b)


def _attn_kernel(h_ref, wq_ref, bq_ref, wk_ref, bk_ref, wv_ref, bv_ref,
                 wo_ref, bo_ref, g_ref, b_ref, o_ref, ctx_ref, qs_ref):
    hd = pl.program_id(0)
    hbf = h_ref[...].astype(_BF)
    q = jnp.dot(hbf, wq_ref[0].astype(_BF), preferred_element_type=_F32) + bq_ref[0]
    k = jnp.dot(hbf, wk_ref[0].astype(_BF), preferred_element_type=_F32) + bk_ref[0]
    v = jnp.dot(hbf, wv_ref[0].astype(_BF), preferred_element_type=_F32) + bv_ref[0]
    qs_ref[...] = (q * (1.0 / math.sqrt(DH))).astype(_BF)
    kb = k.astype(_BF)
    vb = v.astype(_BF)

    def body(i, _):
        qi = qs_ref[pl.ds(i * QB, QB), :]
        s = jax.lax.dot_general(qi, kb, (((1,), (1,)), ((), ())),
                                preferred_element_type=_F32)
        s = s - jnp.max(s, axis=-1, keepdims=True)
        e = jnp.exp(s)
        p = (e / jnp.sum(e, axis=-1, keepdims=True)).astype(_BF)
        ctx_ref[pl.ds(i * QB, QB), :] = jnp.dot(p, vb, preferred_element_type=_F32)
        return 0

    jax.lax.fori_loop(0, T // QB, body, 0)
    contrib = jnp.dot(ctx_ref[...].astype(_BF), wo_ref[0].astype(_BF),
                      preferred_element_type=_F32)

    @pl.when(hd == 0)
    def _():
        o_ref[...] = contrib

    @pl.when(jnp.logical_and(hd > 0, hd < NH - 1))
    def _():
        o_ref[...] = o_ref[...] + contrib

    @pl.when(hd == NH - 1)
    def _():
        out = o_ref[...] + contrib + bo_ref[...] + h_ref[...]
        m = jnp.mean(out, axis=-1, keepdims=True)
        vv = jnp.mean((out - m) ** 2, axis=-1, keepdims=True)
        o_ref[...] = (out - m) / jnp.sqrt(vv + LNEPS) * g_ref[...] + b_ref[...]


def _attn(h, wq, bq, wk, bk, wv, bv, wo, bo, g, b):
    return pl.pallas_call(
        _attn_kernel,
        grid=(NH,),
        in_specs=[
            pl.BlockSpec((T, D), lambda i: (0, 0)),
            pl.BlockSpec((1, D, DH), lambda i: (i, 0, 0)),
            pl.BlockSpec((1, 1, DH), lambda i: (i, 0, 0)),
            pl.BlockSpec((1, D, DH), lambda i: (i, 0, 0)),
            pl.BlockSpec((1, 1, DH), lambda i: (i, 0, 0)),
            pl.BlockSpec((1, D, DH), lambda i: (i, 0, 0)),
            pl.BlockSpec((1, 1, DH), lambda i: (i, 0, 0)),
            pl.BlockSpec((1, DH, D), lambda i: (i, 0, 0)),
            pl.BlockSpec((1, D), lambda i: (0, 0)),
            pl.BlockSpec((1, D), lambda i: (0, 0)),
            pl.BlockSpec((1, D), lambda i: (0, 0)),
        ],
        out_specs=pl.BlockSpec((T, D), lambda i: (0, 0)),
        out_shape=jax.ShapeDtypeStruct((T, D), _F32),
        scratch_shapes=[pltpu.VMEM((T, DH), _F32), pltpu.VMEM((T, DH), _BF)],
        compiler_params=pltpu.CompilerParams(dimension_semantics=("arbitrary",)),
    )(h, wq, bq, wk, bk, wv, bv, wo, bo, g, b)


def _router_kernel(a_ref, wr_ref, eidx_ref, slot_ref, scale_ref, loss_ref):
    logits = jnp.dot(a_ref[...], wr_ref[...], preferred_element_type=_F32)
    m = jnp.max(logits, axis=-1, keepdims=True)
    ex = jnp.exp(logits - m)
    se = jnp.sum(ex, axis=-1, keepdims=True)
    probs = ex / se
    ii = jax.lax.broadcasted_iota(jnp.int32, (T, NE), 1)
    eidx = jnp.min(jnp.where(logits == m, ii, NE), axis=-1, keepdims=True)
    gate = 1.0 / se
    onehot = (ii == eidx).astype(jnp.int32)
    cs = onehot
    sh = 1
    while sh < T:
        cs = cs + jnp.concatenate(
            [jnp.zeros((sh, NE), jnp.int32), cs[: T - sh, :]], axis=0)
        sh *= 2
    pos = jnp.sum(cs * onehot, axis=-1, keepdims=True) - 1
    keep = pos < CAP
    slot = jnp.where(keep, pos, CAP)
    eidx_ref[...] = eidx
    slot_ref[...] = slot
    scale_ref[...] = gate * keep.astype(_F32)
    f = jnp.mean(onehot.astype(_F32), axis=0, keepdims=True)
    pm = jnp.mean(probs, axis=0, keepdims=True)
    loss_ref[...] = jnp.sum(f * pm, axis=-1, keepdims=True) * NE


def _router(a, wr):
    return pl.pallas_call(
        _router_kernel,
        out_shape=(
            jax.ShapeDtypeStruct((T, 1), jnp.int32),
            jax.ShapeDtypeStruct((T, 1), jnp.int32),
            jax.ShapeDtypeStruct((T, 1), _F32),
            jax.ShapeDtypeStruct((1, 1), _F32),
        ),
    )(a, wr)


def _dispatch_kernel(x_ref, eidx_ref, slot_ref, o_ref):
    g = pl.program_id(0)
    eidx = eidx_ref[...]
    slot = slot_ref[...]
    code = jnp.where(slot < CAP, eidx * CAP + slot, -1)
    cols = jax.lax.broadcasted_iota(jnp.int32, (T, SB), 1) + g * SB
    p = (code == cols).astype(_BF)
    o_ref[...] = jax.lax.dot_general(
        p, x_ref[...].astype(_BF), (((0,), (0,)), ((), ())),
        preferred_element_type=_F32)


def _dispatch(x, eidx, slot):
    return pl.pallas_call(
        _dispatch_kernel,
        grid=(NE * CAP // SB,),
        in_specs=[
            pl.BlockSpec((T, D), lambda i: (0, 0)),
            pl.BlockSpec((T, 1), lambda i: (0, 0)),
            pl.BlockSpec((T, 1), lambda i: (0, 0)),
        ],
        out_specs=pl.BlockSpec((SB, D), lambda i: (i, 0)),
        out_shape=jax.ShapeDtypeStruct((NE * CAP, D), _F32),
        compiler_params=pltpu.CompilerParams(dimension_semantics=("arbitrary",)),
    )(x, eidx, slot)


def _ffn_kernel(buf_ref, w1_ref, b1_ref, w2_ref, b2_ref, o_ref):
    xb = buf_ref[...].astype(_BF)
    hh = jnp.dot(xb, w1_ref[0].astype(_BF), preferred_element_type=_F32) + b1_ref[0]
    hh = jax.nn.gelu(hh)
    y = jnp.dot(hh.astype(_BF), w2_ref[0].astype(_BF),
                preferred_element_type=_F32) + b2_ref[0]
    o_ref[...] = y.astype(_BF)


def _ffn(buf, w1, b1, w2, b2):
    return pl.pallas_call(
        _ffn_kernel,
        grid=(NE,),
        in_specs=[
            pl.BlockSpec((CAP, D), lambda e: (e, 0)),
            pl.BlockSpec((1, D, DFF), lambda e: (e, 0, 0)),
            pl.BlockSpec((1, 1, DFF), lambda e: (e, 0, 0)),
            pl.BlockSpec((1, DFF, D), lambda e: (e, 0, 0)),
            pl.BlockSpec((1, 1, D), lambda e: (e, 0, 0)),
        ],
        out_specs=pl.BlockSpec((CAP, D), lambda e: (e, 0)),
        out_shape=jax.ShapeDtypeStruct((NE * CAP, D), _BF),
        compiler_params=pltpu.CompilerParams(dimension_semantics=("parallel",)),
    )(buf, w1, b1, w2, b2)


def _combine_kernel(y_ref, eidx_ref, slot_ref, scale_ref, a_ref, g_ref, b_ref,
                    o_ref):
    t0 = pl.program_id(0)
    code = eidx_ref[...] * CAP + jnp.clip(slot_ref[...], 0, CAP - 1)
    rows = jax.lax.broadcasted_iota(jnp.int32, (TB, NE * CAP), 1)
    p = (code == rows).astype(_BF)
    ffn = jnp.dot(p, y_ref[...], preferred_element_type=_F32) * scale_ref[...]
    out = ffn + a_ref[...]
    m = jnp.mean(out, axis=-1, keepdims=True)
    vv = jnp.mean((out - m) ** 2, axis=-1, keepdims=True)
    o_ref[...] = (out - m) / jnp.sqrt(vv + LNEPS) * g_ref[...] + b_ref[...]


def _combine(y, eidx, slot, scale, a, g, b):
    return pl.pallas_call(
        _combine_kernel,
        grid=(T // TB,),
        in_specs=[
            pl.BlockSpec((NE * CAP, D), lambda i: (0, 0)),
            pl.BlockSpec((TB, 1), lambda i: (i, 0)),
            pl.BlockSpec((TB, 1), lambda i: (i, 0)),
            pl.BlockSpec((TB, 1), lambda i: (i, 0)),
            pl.BlockSpec((TB, D), lambda i: (i, 0)),
            pl.BlockSpec((1, D), lambda i: (0, 0)),
            pl.BlockSpec((1, D), lambda i: (0, 0)),
        ],
        out_specs=pl.BlockSpec((TB, D), lambda i: (i, 0)),
        out_shape=jax.ShapeDtypeStruct((T, D), _F32),
        compiler_params=pltpu.CompilerParams(dimension_semantics=("arbitrary",)),
    )(y, eidx, slot, scale, a, g, b)


def kernel(hidden_states, W_emb, b_emb, Wq, bq, Wk, bk, Wv, bv, Wo, bo,
           ln_attn_g, ln_attn_b, Wr, W1, b1, W2, b2, ln_out_g, ln_out_b):
    x = hidden_states.reshape(T, EMBD)
    r2 = lambda v: v.reshape(1, -1)
    wqh = Wq.reshape(D, NH, DH).transpose(1, 0, 2)
    wkh = Wk.reshape(D, NH, DH).transpose(1, 0, 2)
    wvh = Wv.reshape(D, NH, DH).transpose(1, 0, 2)
    woh = Wo.reshape(NH, DH, D)
    bqh = bq.reshape(NH, 1, DH)
    bkh = bk.reshape(NH, 1, DH)
    bvh = bv.reshape(NH, 1, DH)
    b1h = b1.reshape(NE, 1, DFF)
    b2h = b2.reshape(NE, 1, D)
    h = _embed(x, W_emb, r2(b_emb))
    losses = []
    for _ in range(NLAYERS):
        a = _attn(h, wqh, bqh, wkh, bkh, wvh, bvh, woh, r2(bo),
                  r2(ln_attn_g), r2(ln_attn_b))
        eidx, slot, scale, loss = _router(a, Wr)
        buf = _dispatch(a, eidx, slot)
        y = _ffn(buf, W1, b1h, W2, b2h)
        h = _combine(y, eidx, slot, scale, a, r2(ln_out_g), r2(ln_out_b))
        losses.append(loss[0, 0])
    return h.reshape(1, T, D), jnp.stack(losses)


# speedup vs baseline: 1.2531x; 1.1294x over previous
import math

import jax
import jax.numpy as jnp
from jax.experimental import pallas as pl
from jax.experimental.pallas import tpu as pltpu

EMBD = 128
D = 1024
NH = 16
DH = 64
NE = 64
DFF = 2048
NLAYERS = 2
LNEPS = 1e-12
T = 2048
CAP = 64
NSLOT = NE * CAP
QB = 256
SB = 512
TB = 256
HB = 128

_BF = jnp.bfloat16
_F32 = jnp.float32


def _ln(x, g, b):
    m = jnp.mean(x, axis=-1, keepdims=True)
    v = jnp.mean((x - m) ** 2, axis=-1, keepdims=True)
    return (x - m) / jnp.sqrt(v + LNEPS) * g + b


def _embed_kernel(x_ref, w_ref, b_ref, o_ref, obf_ref):
    h = jnp.dot(x_ref[...], w_ref[...], preferred_element_type=_F32) + b_ref[...]
    o_ref[...] = h
    obf_ref[...] = h.astype(_BF)


def _embed(x, w, b):
    return pl.pallas_call(
        _embed_kernel,
        out_shape=(
            jax.ShapeDtypeStruct((T, D), _F32),
            jax.ShapeDtypeStruct((T, D), _BF),
        ),
    )(x, w, b)


def _qkv_kernel(hbf_ref, wq_ref, bq_ref, wk_ref, bk_ref, wv_ref, bv_ref,
                q_ref, k_ref, v_ref):
    hbf = hbf_ref[...]
    q = jnp.dot(hbf, wq_ref[...].astype(_BF), preferred_element_type=_F32) + bq_ref[...]
    k = jnp.dot(hbf, wk_ref[...].astype(_BF), preferred_element_type=_F32) + bk_ref[...]
    v = jnp.dot(hbf, wv_ref[...].astype(_BF), preferred_element_type=_F32) + bv_ref[...]
    q_ref[...] = (q * (1.0 / math.sqrt(DH))).astype(_BF)
    k_ref[...] = k.astype(_BF)
    v_ref[...] = v.astype(_BF)


def _qkv(hbf, wq, bq, wk, bk, wv, bv):
    return pl.pallas_call(
        _qkv_kernel,
        grid=(D // HB,),
        in_specs=[
            pl.BlockSpec((T, D), lambda i: (0, 0)),
            pl.BlockSpec((D, HB), lambda i: (0, i)),
            pl.BlockSpec((1, HB), lambda i: (0, i)),
            pl.BlockSpec((D, HB), lambda i: (0, i)),
            pl.BlockSpec((1, HB), lambda i: (0, i)),
            pl.BlockSpec((D, HB), lambda i: (0, i)),
            pl.BlockSpec((1, HB), lambda i: (0, i)),
        ],
        out_specs=(
            pl.BlockSpec((T, HB), lambda i: (0, i)),
            pl.BlockSpec((T, HB), lambda i: (0, i)),
            pl.BlockSpec((T, HB), lambda i: (0, i)),
        ),
        out_shape=(
            jax.ShapeDtypeStruct((T, D), _BF),
            jax.ShapeDtypeStruct((T, D), _BF),
            jax.ShapeDtypeStruct((T, D), _BF),
        ),
        compiler_params=pltpu.CompilerParams(dimension_semantics=("arbitrary",)),
    )(hbf, wq, bq, wk, bk, wv, bv)


def _flash_kernel(q_ref, k_ref, v_ref, ctx_ref):
    for sub in range(2):
        lo = sub * DH
        kh = k_ref[:, lo:lo + DH]
        vh = v_ref[:, lo:lo + DH]

        def body(i, _):
            qh = q_ref[pl.ds(i * QB, QB), lo:lo + DH]
            s = jax.lax.dot_general(qh, kh, (((1,), (1,)), ((), ())),
                                    preferred_element_type=_F32)
            s = s - jnp.max(s, axis=-1, keepdims=True)
            e = jnp.exp(s)
            se = jnp.sum(e, axis=-1, keepdims=True)
            cb = jnp.dot(e.astype(_BF), vh, preferred_element_type=_F32)
            ctx_ref[pl.ds(i * QB, QB), lo:lo + DH] = (cb / se).astype(_BF)
            return 0

        jax.lax.fori_loop(0, T // QB, body, 0)


def _flash(q, k, v):
    return pl.pallas_call(
        _flash_kernel,
        grid=(D // HB,),
        in_specs=[
            pl.BlockSpec((T, HB), lambda i: (0, i)),
            pl.BlockSpec((T, HB), lambda i: (0, i)),
            pl.BlockSpec((T, HB), lambda i: (0, i)),
        ],
        out_specs=pl.BlockSpec((T, HB), lambda i: (0, i)),
        out_shape=jax.ShapeDtypeStruct((T, D), _BF),
        compiler_params=pltpu.CompilerParams(dimension_semantics=("arbitrary",)),
    )(q, k, v)


def _router_kernel(ctx_ref, wo_ref, bo_ref, h_ref, g_ref, b_ref, wr_ref,
                   a_ref, abf_ref, code_ref, code2_ref, scale_ref, loss_ref):
    att = jnp.dot(ctx_ref[...], wo_ref[...].astype(_BF),
                  preferred_element_type=_F32) + bo_ref[...] + h_ref[...]
    a = _ln(att, g_ref[...], b_ref[...])
    a_ref[...] = a
    abf_ref[...] = a.astype(_BF)
    logits = jnp.dot(a, wr_ref[...], preferred_element_type=_F32)
    m = jnp.max(logits, axis=-1, keepdims=True)
    ex = jnp.exp(logits - m)
    se = jnp.sum(ex, axis=-1, keepdims=True)
    ii = jax.lax.broadcasted_iota(jnp.int32, (T, NE), 1)
    eidx = jnp.min(jnp.where(logits == m, ii, NE), axis=-1, keepdims=True)
    gate = 1.0 / se
    onehot = (ii == eidx).astype(jnp.int32)
    cs = onehot
    sh = 1
    while sh < T:
        cs = cs + jnp.concatenate(
            [jnp.zeros((sh, NE), jnp.int32), cs[: T - sh, :]], axis=0)
        sh *= 2
    pos = jnp.sum(cs * onehot, axis=-1, keepdims=True) - 1
    keep = pos < CAP
    slot = jnp.where(keep, pos, CAP)
    code_ref[...] = jnp.where(keep, eidx * CAP + slot, NSLOT)
    code2_ref[...] = eidx * CAP + jnp.clip(slot, 0, CAP - 1)
    scale_ref[...] = gate * keep.astype(_F32)
    probs = ex / se
    f = jnp.mean(onehot.astype(_F32), axis=0, keepdims=True)
    pm = jnp.mean(probs, axis=0, keepdims=True)
    loss_ref[...] = jnp.sum(f * pm, axis=-1, keepdims=True) * NE


def _router(ctx, wo, bo, h, g, b, wr):
    return pl.pallas_call(
        _router_kernel,
        out_shape=(
            jax.ShapeDtypeStruct((T, D), _F32),
            jax.ShapeDtypeStruct((T, D), _BF),
            jax.ShapeDtypeStruct((T, 1), jnp.int32),
            jax.ShapeDtypeStruct((T, 1), jnp.int32),
            jax.ShapeDtypeStruct((T, 1), _F32),
            jax.ShapeDtypeStruct((1, 1), _F32),
        ),
    )(ctx, wo, bo, h, g, b, wr)


def _dispatch_kernel(xbf_ref, code_ref, o_ref):
    g = pl.program_id(0)
    code = code_ref[...]
    cols = jax.lax.broadcasted_iota(jnp.int32, (T, SB), 1) + g * SB
    p = (code == cols).astype(_BF)
    o_ref[...] = jax.lax.dot_general(
        p, xbf_ref[...], (((0,), (0,)), ((), ())), preferred_element_type=_F32)


def _dispatch(xbf, code):
    return pl.pallas_call(
        _dispatch_kernel,
        grid=(NSLOT // SB,),
        in_specs=[
            pl.BlockSpec((T, D), lambda i: (0, 0)),
            pl.BlockSpec((T, 1), lambda i: (0, 0)),
        ],
        out_specs=pl.BlockSpec((SB, D), lambda i: (i, 0)),
        out_shape=jax.ShapeDtypeStruct((NSLOT, D), _F32),
        compiler_params=pltpu.CompilerParams(dimension_semantics=("arbitrary",)),
    )(xbf, code)


def _ffn_kernel(buf_ref, w1_ref, b1_ref, w2_ref, b2_ref, o_ref):
    xb = buf_ref[...].astype(_BF)
    hh = jnp.dot(xb, w1_ref[0].astype(_BF), preferred_element_type=_F32) + b1_ref[0]
    hh = jax.nn.gelu(hh)
    y = jnp.dot(hh.astype(_BF), w2_ref[0].astype(_BF),
                preferred_element_type=_F32) + b2_ref[0]
    o_ref[...] = y.astype(_BF)


def _ffn(buf, w1, b1, w2, b2):
    return pl.pallas_call(
        _ffn_kernel,
        grid=(NE,),
        in_specs=[
            pl.BlockSpec((CAP, D), lambda e: (e, 0)),
            pl.BlockSpec((1, D, DFF), lambda e: (e, 0, 0)),
            pl.BlockSpec((1, 1, DFF), lambda e: (e, 0, 0)),
            pl.BlockSpec((1, DFF, D), lambda e: (e, 0, 0)),
            pl.BlockSpec((1, 1, D), lambda e: (e, 0, 0)),
        ],
        out_specs=pl.BlockSpec((CAP, D), lambda e: (e, 0)),
        out_shape=jax.ShapeDtypeStruct((NSLOT, D), _BF),
        compiler_params=pltpu.CompilerParams(dimension_semantics=("parallel",)),
    )(buf, w1, b1, w2, b2)


def _combine_kernel(y_ref, code2_ref, scale_ref, a_ref, g_ref, b_ref,
                    o_ref, obf_ref):
    code2 = code2_ref[...]
    rows = jax.lax.broadcasted_iota(jnp.int32, (TB, NSLOT), 1)
    p = (code2 == rows).astype(_BF)
    ffn = jnp.dot(p, y_ref[...], preferred_element_type=_F32) * scale_ref[...]
    out = _ln(ffn + a_ref[...], g_ref[...], b_ref[...])
    o_ref[...] = out
    obf_ref[...] = out.astype(_BF)


def _combine(y, code2, scale, a, g, b):
    return pl.pallas_call(
        _combine_kernel,
        grid=(T // TB,),
        in_specs=[
            pl.BlockSpec((NSLOT, D), lambda i: (0, 0)),
            pl.BlockSpec((TB, 1), lambda i: (i, 0)),
            pl.BlockSpec((TB, 1), lambda i: (i, 0)),
            pl.BlockSpec((TB, D), lambda i: (i, 0)),
            pl.BlockSpec((1, D), lambda i: (0, 0)),
            pl.BlockSpec((1, D), lambda i: (0, 0)),
        ],
        out_specs=(
            pl.BlockSpec((TB, D), lambda i: (i, 0)),
            pl.BlockSpec((TB, D), lambda i: (i, 0)),
        ),
        out_shape=(
            jax.ShapeDtypeStruct((T, D), _F32),
            jax.ShapeDtypeStruct((T, D), _BF),
        ),
        compiler_params=pltpu.CompilerParams(dimension_semantics=("arbitrary",)),
    )(y, code2, scale, a, g, b)


def kernel(hidden_states, W_emb, b_emb, Wq, bq, Wk, bk, Wv, bv, Wo, bo,
           ln_attn_g, ln_attn_b, Wr, W1, b1, W2, b2, ln_out_g, ln_out_b):
    x = hidden_states.reshape(T, EMBD)
    r2 = lambda v: v.reshape(1, -1)
    b1h = b1.reshape(NE, 1, DFF)
    b2h = b2.reshape(NE, 1, D)
    h, hbf = _embed(x, W_emb, r2(b_emb))
    losses = []
    for _ in range(NLAYERS):
        q, k, v = _qkv(hbf, Wq, r2(bq), Wk, r2(bk), Wv, r2(bv))
        ctx = _flash(q, k, v)
        a, abf, code, code2, scale, loss = _router(
            ctx, Wo, r2(bo), h, r2(ln_attn_g), r2(ln_attn_b), Wr)
        buf = _dispatch(abf, code)
        y = _ffn(buf, W1, b1h, W2, b2h)
        h, hbf = _combine(y, code2, scale, a, r2(ln_out_g), r2(ln_out_b))
        losses.append(loss[0, 0])
    return h.reshape(1, T, D), jnp.stack(losses)


# speedup vs baseline: 1.2562x; 1.0024x over previous
import functools
import math

import jax
import jax.numpy as jnp
from jax import lax
from jax.experimental import pallas as pl
from jax.experimental.pallas import tpu as pltpu
from jax.experimental.pallas import tpu_sc as plsc

EMBD = 128
D = 1024
NH = 16
DH = 64
NE = 64
DFF = 2048
NLAYERS = 2
LNEPS = 1e-12
T = 2048
CAP = 64
NSLOT = NE * CAP
QB = 256
TB = 256
HB = 128
NSLOTP = NSLOT + 128

_BF = jnp.bfloat16
_F32 = jnp.float32


def _ln(x, g, b):
    m = jnp.mean(x, axis=-1, keepdims=True)
    v = jnp.mean((x - m) ** 2, axis=-1, keepdims=True)
    return (x - m) / jnp.sqrt(v + LNEPS) * g + b


def _embed_kernel(x_ref, w_ref, b_ref, o_ref, obf_ref):
    h = jnp.dot(x_ref[...], w_ref[...], preferred_element_type=_F32) + b_ref[...]
    o_ref[...] = h
    obf_ref[...] = h.astype(_BF)


def _embed(x, w, b):
    return pl.pallas_call(
        _embed_kernel,
        out_shape=(
            jax.ShapeDtypeStruct((T, D), _F32),
            jax.ShapeDtypeStruct((T, D), _BF),
        ),
    )(x, w, b)


def _qkv_kernel(hbf_ref, wq_ref, bq_ref, wk_ref, bk_ref, wv_ref, bv_ref,
                q_ref, k_ref, v_ref):
    hbf = hbf_ref[...]
    q = jnp.dot(hbf, wq_ref[...].astype(_BF), preferred_element_type=_F32) + bq_ref[...]
    k = jnp.dot(hbf, wk_ref[...].astype(_BF), preferred_element_type=_F32) + bk_ref[...]
    v = jnp.dot(hbf, wv_ref[...].astype(_BF), preferred_element_type=_F32) + bv_ref[...]
    q_ref[...] = (q * (1.0 / math.sqrt(DH))).astype(_BF)
    k_ref[...] = k.astype(_BF)
    v_ref[...] = v.astype(_BF)


def _qkv(hbf, wq, bq, wk, bk, wv, bv):
    return pl.pallas_call(
        _qkv_kernel,
        grid=(D // HB,),
        in_specs=[
            pl.BlockSpec((T, D), lambda i: (0, 0)),
            pl.BlockSpec((D, HB), lambda i: (0, i)),
            pl.BlockSpec((1, HB), lambda i: (0, i)),
            pl.BlockSpec((D, HB), lambda i: (0, i)),
            pl.BlockSpec((1, HB), lambda i: (0, i)),
            pl.BlockSpec((D, HB), lambda i: (0, i)),
            pl.BlockSpec((1, HB), lambda i: (0, i)),
        ],
        out_specs=(
            pl.BlockSpec((T, HB), lambda i: (0, i)),
            pl.BlockSpec((T, HB), lambda i: (0, i)),
            pl.BlockSpec((T, HB), lambda i: (0, i)),
        ),
        out_shape=(
            jax.ShapeDtypeStruct((T, D), _BF),
            jax.ShapeDtypeStruct((T, D), _BF),
            jax.ShapeDtypeStruct((T, D), _BF),
        ),
        compiler_params=pltpu.CompilerParams(dimension_semantics=("arbitrary",)),
    )(hbf, wq, bq, wk, bk, wv, bv)


def _flash_kernel(q_ref, k_ref, v_ref, ctx_ref):
    for sub in range(2):
        lo = sub * DH
        kh = k_ref[:, lo:lo + DH]
        vh = v_ref[:, lo:lo + DH]

        def body(i, _):
            qh = q_ref[pl.ds(i * QB, QB), lo:lo + DH]
            s = jax.lax.dot_general(qh, kh, (((1,), (1,)), ((), ())),
                                    preferred_element_type=_F32)
            s = s - jnp.max(s, axis=-1, keepdims=True)
            e = jnp.exp(s.astype(_BF))
            se = jnp.sum(e, axis=-1, keepdims=True).astype(_F32)
            cb = jnp.dot(e, vh, preferred_element_type=_F32)
            ctx_ref[pl.ds(i * QB, QB), lo:lo + DH] = (cb / se).astype(_BF)
            return 0

        jax.lax.fori_loop(0, T // QB, body, 0)


def _flash(q, k, v):
    return pl.pallas_call(
        _flash_kernel,
        grid=(D // HB,),
        in_specs=[
            pl.BlockSpec((T, HB), lambda i: (0, i)),
            pl.BlockSpec((T, HB), lambda i: (0, i)),
            pl.BlockSpec((T, HB), lambda i: (0, i)),
        ],
        out_specs=pl.BlockSpec((T, HB), lambda i: (0, i)),
        out_shape=jax.ShapeDtypeStruct((T, D), _BF),
        compiler_params=pltpu.CompilerParams(dimension_semantics=("arbitrary",)),
    )(q, k, v)


def _router_kernel(ctx_ref, wo_ref, bo_ref, h_ref, g_ref, b_ref, wr_ref,
                   a_ref, code_ref, code2_ref, scale_ref, loss_ref):
    att = jnp.dot(ctx_ref[...], wo_ref[...].astype(_BF),
                  preferred_element_type=_F32) + bo_ref[...] + h_ref[...]
    a = _ln(att, g_ref[...], b_ref[...])
    a_ref[...] = a
    logits = jnp.dot(a, wr_ref[...], preferred_element_type=_F32)
    m = jnp.max(logits, axis=-1, keepdims=True)
    ex = jnp.exp(logits - m)
    se = jnp.sum(ex, axis=-1, keepdims=True)
    ii = jax.lax.broadcasted_iota(jnp.int32, (T, NE), 1)
    eidx = jnp.min(jnp.where(logits == m, ii, NE), axis=-1, keepdims=True)
    gate = 1.0 / se
    onehot = (ii == eidx).astype(jnp.int32)
    cs = onehot
    sh = 1
    while sh < T:
        cs = cs + jnp.concatenate(
            [jnp.zeros((sh, NE), jnp.int32), cs[: T - sh, :]], axis=0)
        sh *= 2
    pos = jnp.sum(cs * onehot, axis=-1, keepdims=True) - 1
    keep = pos < CAP
    slot = jnp.where(keep, pos, CAP)
    tt = jax.lax.broadcasted_iota(jnp.int32, (T, 1), 0)
    code_ref[...] = jnp.where(keep, eidx * CAP + slot, NSLOT + (tt % 128))
    code2_ref[...] = eidx * CAP + jnp.clip(slot, 0, CAP - 1)
    scale_ref[...] = gate * keep.astype(_F32)
    probs = ex / se
    f = jnp.mean(onehot.astype(_F32), axis=0, keepdims=True)
    pm = jnp.mean(probs, axis=0, keepdims=True)
    loss_ref[...] = jnp.sum(f * pm, axis=-1, keepdims=True) * NE


def _router(ctx, wo, bo, h, g, b, wr):
    return pl.pallas_call(
        _router_kernel,
        out_shape=(
            jax.ShapeDtypeStruct((T, D), _F32),
            jax.ShapeDtypeStruct((T, 1), jnp.int32),
            jax.ShapeDtypeStruct((T, 1), jnp.int32),
            jax.ShapeDtypeStruct((T, 1), _F32),
            jax.ShapeDtypeStruct((1, 1), _F32),
        ),
    )(ctx, wo, bo, h, g, b, wr)


def _wid():
    return lax.axis_index("s") * 2 + lax.axis_index("c")


_DCH = 64
_sc_cache = {}


def _sc_kernels():
    if _sc_cache:
        return _sc_cache["disp"], _sc_cache["comb"]
    mesh = plsc.VectorSubcoreMesh(core_axis_name="c", subcore_axis_name="s")

    @functools.partial(
        pl.kernel,
        out_type=jax.ShapeDtypeStruct((NSLOTP, D), _F32),
        mesh=mesh,
        scratch_types=[
            pltpu.VMEM((T // 32,), jnp.int32),
            pltpu.VMEM((T // 32, D), _F32),
            pltpu.SemaphoreType.DMA,
        ],
    )
    def disp(a_hbm, code_hbm, buf_hbm, idx_v, rows_v, sem):
        base = _wid() * (T // 32)
        pltpu.sync_copy(code_hbm.at[pl.ds(base, T // 32)], idx_v)
        pltpu.sync_copy(a_hbm.at[pl.ds(base, T // 32)], rows_v)
        pltpu.async_copy(rows_v, buf_hbm.at[idx_v], sem).wait()

    @functools.partial(
        pl.kernel,
        out_type=jax.ShapeDtypeStruct((T, D), _F32),
        mesh=mesh,
        scratch_types=[
            pltpu.VMEM((T // 32,), jnp.int32),
            pltpu.VMEM((T // 32, D), _F32),
            pltpu.SemaphoreType.DMA,
        ],
    )
    def comb(y_hbm, code2_hbm, out_hbm, idx_v, rows_v, sem):
        base = _wid() * (T // 32)
        pltpu.sync_copy(code2_hbm.at[pl.ds(base, T // 32)], idx_v)
        pltpu.async_copy(y_hbm.at[idx_v], rows_v, sem).wait()
        pltpu.sync_copy(rows_v, out_hbm.at[pl.ds(base, T // 32)])

    _sc_cache.update(disp=disp, comb=comb)
    return disp, comb


def _ffn_kernel(buf_ref, w1_ref, b1_ref, w2_ref, b2_ref, o_ref):
    xb = buf_ref[...].astype(_BF)
    hh = jnp.dot(xb, w1_ref[0].astype(_BF), preferred_element_type=_F32) + b1_ref[0]
    hh = jax.nn.gelu(hh)
    o_ref[...] = jnp.dot(hh.astype(_BF), w2_ref[0].astype(_BF),
                         preferred_element_type=_F32) + b2_ref[0]


def _ffn(buf, w1, b1, w2, b2):
    return pl.pallas_call(
        _ffn_kernel,
        grid=(NE,),
        in_specs=[
            pl.BlockSpec((CAP, D), lambda e: (e, 0)),
            pl.BlockSpec((1, D, DFF), lambda e: (e, 0, 0)),
            pl.BlockSpec((1, 1, DFF), lambda e: (e, 0, 0)),
            pl.BlockSpec((1, DFF, D), lambda e: (e, 0, 0)),
            pl.BlockSpec((1, 1, D), lambda e: (e, 0, 0)),
        ],
        out_specs=pl.BlockSpec((CAP, D), lambda e: (e, 0)),
        out_shape=jax.ShapeDtypeStruct((NSLOT, D), _F32),
        compiler_params=pltpu.CompilerParams(dimension_semantics=("parallel",)),
    )(buf, w1, b1, w2, b2)


def _lnout_kernel(gath_ref, scale_ref, a_ref, g_ref, b_ref, o_ref, obf_ref):
    ffn = gath_ref[...] * scale_ref[...]
    out = _ln(ffn + a_ref[...], g_ref[...], b_ref[...])
    o_ref[...] = out
    obf_ref[...] = out.astype(_BF)


def _lnout(gath, scale, apad, g, b):
    return pl.pallas_call(
        _lnout_kernel,
        grid=(T // TB,),
        in_specs=[
            pl.BlockSpec((TB, D), lambda i: (i, 0)),
            pl.BlockSpec((TB, 1), lambda i: (i, 0)),
            pl.BlockSpec((TB, D), lambda i: (i, 0)),
            pl.BlockSpec((1, D), lambda i: (0, 0)),
            pl.BlockSpec((1, D), lambda i: (0, 0)),
        ],
        out_specs=(
            pl.BlockSpec((TB, D), lambda i: (i, 0)),
            pl.BlockSpec((TB, D), lambda i: (i, 0)),
        ),
        out_shape=(
            jax.ShapeDtypeStruct((T, D), _F32),
            jax.ShapeDtypeStruct((T, D), _BF),
        ),
        compiler_params=pltpu.CompilerParams(dimension_semantics=("arbitrary",)),
    )(gath, scale, apad, g, b)


def kernel(hidden_states, W_emb, b_emb, Wq, bq, Wk, bk, Wv, bv, Wo, bo,
           ln_attn_g, ln_attn_b, Wr, W1, b1, W2, b2, ln_out_g, ln_out_b):
    x = hidden_states.reshape(T, EMBD)
    r2 = lambda v: v.reshape(1, -1)
    b1h = b1.reshape(NE, 1, DFF)
    b2h = b2.reshape(NE, 1, D)
    h, hbf = _embed(x, W_emb, r2(b_emb))
    losses = []
    for _ in range(NLAYERS):
        q, k, v = _qkv(hbf, Wq, r2(bq), Wk, r2(bk), Wv, r2(bv))
        ctx = _flash(q, k, v)
        a, code, code2, scale, loss = _router(
            ctx, Wo, r2(bo), h, r2(ln_attn_g), r2(ln_attn_b), Wr)
        sc_dispatch, sc_combine = _sc_kernels()
        buf = sc_dispatch(a, code.reshape(T))
        y = _ffn(buf, W1, b1h, W2, b2h)
        gath = sc_combine(y, code2.reshape(T))
        h, hbf = _lnout(gath, scale, a, r2(ln_out_g), r2(ln_out_b))
        losses.append(loss[0, 0])
    return h.reshape(1, T, D), jnp.stack(losses)


# speedup vs baseline: 1.2597x; 1.0028x over previous
import functools
import math

import jax
import jax.numpy as jnp
from jax import lax
from jax.experimental import pallas as pl
from jax.experimental.pallas import tpu as pltpu
from jax.experimental.pallas import tpu_sc as plsc

EMBD = 128
D = 1024
NH = 16
DH = 64
NE = 64
DFF = 2048
NLAYERS = 2
LNEPS = 1e-12
T = 2048
CAP = 64
NSLOT = NE * CAP
QB = 256
TB = 256
HB = 128
NSLOTP = NSLOT + 128

_BF = jnp.bfloat16
_F32 = jnp.float32


def _ln(x, g, b):
    m = jnp.mean(x, axis=-1, keepdims=True)
    v = jnp.mean((x - m) ** 2, axis=-1, keepdims=True)
    return (x - m) / jnp.sqrt(v + LNEPS) * g + b


def _embed_kernel(x_ref, w_ref, b_ref, o_ref, obf_ref):
    h = jnp.dot(x_ref[...], w_ref[...], preferred_element_type=_F32) + b_ref[...]
    o_ref[...] = h
    obf_ref[...] = h.astype(_BF)


def _embed(x, w, b):
    return pl.pallas_call(
        _embed_kernel,
        out_shape=(
            jax.ShapeDtypeStruct((T, D), _F32),
            jax.ShapeDtypeStruct((T, D), _BF),
        ),
    )(x, w, b)


def _attn_kernel(hbf_ref, wq_ref, bq_ref, wk_ref, bk_ref, wv_ref, bv_ref,
                 ctx_ref, q_sc, k_sc, v_sc):
    hbf = hbf_ref[...]
    q = jnp.dot(hbf, wq_ref[...].astype(_BF), preferred_element_type=_F32) + bq_ref[...]
    k = jnp.dot(hbf, wk_ref[...].astype(_BF), preferred_element_type=_F32) + bk_ref[...]
    v = jnp.dot(hbf, wv_ref[...].astype(_BF), preferred_element_type=_F32) + bv_ref[...]
    q_sc[...] = (q * (1.0 / math.sqrt(DH))).astype(_BF)
    k_sc[...] = k.astype(_BF)
    v_sc[...] = v.astype(_BF)
    for sub in range(2):
        lo = sub * DH
        kh = k_sc[:, lo:lo + DH]
        vh = v_sc[:, lo:lo + DH]

        def body(i, _):
            qh = q_sc[pl.ds(i * QB, QB), lo:lo + DH]
            s = jax.lax.dot_general(qh, kh, (((1,), (1,)), ((), ())),
                                    preferred_element_type=_F32)
            s = s - jnp.max(s, axis=-1, keepdims=True)
            e = jnp.exp(s.astype(_BF))
            se = jnp.sum(e, axis=-1, keepdims=True).astype(_F32)
            cb = jnp.dot(e, vh, preferred_element_type=_F32)
            ctx_ref[pl.ds(i * QB, QB), lo:lo + DH] = (cb / se).astype(_BF)
            return 0

        jax.lax.fori_loop(0, T // QB, body, 0)


def _attn(hbf, wq, bq, wk, bk, wv, bv):
    return pl.pallas_call(
        _attn_kernel,
        grid=(D // HB,),
        in_specs=[
            pl.BlockSpec((T, D), lambda i: (0, 0)),
            pl.BlockSpec((D, HB), lambda i: (0, i)),
            pl.BlockSpec((1, HB), lambda i: (0, i)),
            pl.BlockSpec((D, HB), lambda i: (0, i)),
            pl.BlockSpec((1, HB), lambda i: (0, i)),
            pl.BlockSpec((D, HB), lambda i: (0, i)),
            pl.BlockSpec((1, HB), lambda i: (0, i)),
        ],
        out_specs=pl.BlockSpec((T, HB), lambda i: (0, i)),
        out_shape=jax.ShapeDtypeStruct((T, D), _BF),
        scratch_shapes=[
            pltpu.VMEM((T, HB), _BF),
            pltpu.VMEM((T, HB), _BF),
            pltpu.VMEM((T, HB), _BF),
        ],
        compiler_params=pltpu.CompilerParams(dimension_semantics=("arbitrary",)),
    )(hbf, wq, bq, wk, bk, wv, bv)


def _router_kernel(ctx_ref, wo_ref, bo_ref, h_ref, g_ref, b_ref, wr_ref,
                   a_ref, code_ref, code2_ref, scale_ref, loss_ref):
    att = jnp.dot(ctx_ref[...], wo_ref[...].astype(_BF),
                  preferred_element_type=_F32) + bo_ref[...] + h_ref[...]
    a = _ln(att, g_ref[...], b_ref[...])
    a_ref[...] = a
    logits = jnp.dot(a, wr_ref[...], preferred_element_type=_F32)
    m = jnp.max(logits, axis=-1, keepdims=True)
    ex = jnp.exp(logits - m)
    se = jnp.sum(ex, axis=-1, keepdims=True)
    ii = jax.lax.broadcasted_iota(jnp.int32, (T, NE), 1)
    eidx = jnp.min(jnp.where(logits == m, ii, NE), axis=-1, keepdims=True)
    gate = 1.0 / se
    onehot = (ii == eidx).astype(jnp.int32)
    cs = onehot
    sh = 1
    while sh < T:
        cs = cs + jnp.concatenate(
            [jnp.zeros((sh, NE), jnp.int32), cs[: T - sh, :]], axis=0)
        sh *= 2
    pos = jnp.sum(cs * onehot, axis=-1, keepdims=True) - 1
    keep = pos < CAP
    slot = jnp.where(keep, pos, CAP)
    tt = jax.lax.broadcasted_iota(jnp.int32, (T, 1), 0)
    code_ref[...] = jnp.where(keep, eidx * CAP + slot, NSLOT + (tt % 128))
    code2_ref[...] = eidx * CAP + jnp.clip(slot, 0, CAP - 1)
    scale_ref[...] = gate * keep.astype(_F32)
    probs = ex / se
    f = jnp.mean(onehot.astype(_F32), axis=0, keepdims=True)
    pm = jnp.mean(probs, axis=0, keepdims=True)
    loss_ref[...] = jnp.sum(f * pm, axis=-1, keepdims=True) * NE


def _router(ctx, wo, bo, h, g, b, wr):
    return pl.pallas_call(
        _router_kernel,
        out_shape=(
            jax.ShapeDtypeStruct((T, D), _F32),
            jax.ShapeDtypeStruct((T, 1), jnp.int32),
            jax.ShapeDtypeStruct((T, 1), jnp.int32),
            jax.ShapeDtypeStruct((T, 1), _F32),
            jax.ShapeDtypeStruct((1, 1), _F32),
        ),
    )(ctx, wo, bo, h, g, b, wr)


def _wid():
    return lax.axis_index("s") * 2 + lax.axis_index("c")


_DCH = 64
_sc_cache = {}


def _sc_kernels():
    if _sc_cache:
        return _sc_cache["disp"], _sc_cache["comb"]
    mesh = plsc.VectorSubcoreMesh(core_axis_name="c", subcore_axis_name="s")

    @functools.partial(
        pl.kernel,
        out_type=jax.ShapeDtypeStruct((NSLOTP, D), _F32),
        mesh=mesh,
        scratch_types=[
            pltpu.VMEM((T // 32,), jnp.int32),
            pltpu.VMEM((T // 32, D), _F32),
            pltpu.SemaphoreType.DMA,
        ],
    )
    def disp(a_hbm, code_hbm, buf_hbm, idx_v, rows_v, sem):
        base = _wid() * (T // 32)
        pltpu.sync_copy(code_hbm.at[pl.ds(base, T // 32)], idx_v)
        pltpu.sync_copy(a_hbm.at[pl.ds(base, T // 32)], rows_v)
        pltpu.async_copy(rows_v, buf_hbm.at[idx_v], sem).wait()

    @functools.partial(
        pl.kernel,
        out_type=jax.ShapeDtypeStruct((T, D), _F32),
        mesh=mesh,
        scratch_types=[
            pltpu.VMEM((T // 32,), jnp.int32),
            pltpu.VMEM((T // 32, D), _F32),
            pltpu.SemaphoreType.DMA,
        ],
    )
    def comb(y_hbm, code2_hbm, out_hbm, idx_v, rows_v, sem):
        base = _wid() * (T // 32)
        pltpu.sync_copy(code2_hbm.at[pl.ds(base, T // 32)], idx_v)
        pltpu.async_copy(y_hbm.at[idx_v], rows_v, sem).wait()
        pltpu.sync_copy(rows_v, out_hbm.at[pl.ds(base, T // 32)])

    _sc_cache.update(disp=disp, comb=comb)
    return disp, comb


def _ffn_kernel(buf_ref, w1_ref, b1_ref, w2_ref, b2_ref, o_ref):
    xb = buf_ref[...].astype(_BF)
    hh = jnp.dot(xb, w1_ref[0].astype(_BF), preferred_element_type=_F32) + b1_ref[0]
    hh = jax.nn.gelu(hh)
    o_ref[...] = jnp.dot(hh.astype(_BF), w2_ref[0].astype(_BF),
                         preferred_element_type=_F32) + b2_ref[0]


def _ffn(buf, w1, b1, w2, b2):
    return pl.pallas_call(
        _ffn_kernel,
        grid=(NE,),
        in_specs=[
            pl.BlockSpec((CAP, D), lambda e: (e, 0)),
            pl.BlockSpec((1, D, DFF), lambda e: (e, 0, 0)),
            pl.BlockSpec((1, 1, DFF), lambda e: (e, 0, 0)),
            pl.BlockSpec((1, DFF, D), lambda e: (e, 0, 0)),
            pl.BlockSpec((1, 1, D), lambda e: (e, 0, 0)),
        ],
        out_specs=pl.BlockSpec((CAP, D), lambda e: (e, 0)),
        out_shape=jax.ShapeDtypeStruct((NSLOT, D), _F32),
        compiler_params=pltpu.CompilerParams(dimension_semantics=("parallel",)),
    )(buf, w1, b1, w2, b2)


def _lnout_kernel(gath_ref, scale_ref, a_ref, g_ref, b_ref, o_ref, obf_ref):
    ffn = gath_ref[...] * scale_ref[...]
    out = _ln(ffn + a_ref[...], g_ref[...], b_ref[...])
    o_ref[...] = out
    obf_ref[...] = out.astype(_BF)


def _lnout(gath, scale, apad, g, b):
    return pl.pallas_call(
        _lnout_kernel,
        grid=(T // TB,),
        in_specs=[
            pl.BlockSpec((TB, D), lambda i: (i, 0)),
            pl.BlockSpec((TB, 1), lambda i: (i, 0)),
            pl.BlockSpec((TB, D), lambda i: (i, 0)),
            pl.BlockSpec((1, D), lambda i: (0, 0)),
            pl.BlockSpec((1, D), lambda i: (0, 0)),
        ],
        out_specs=(
            pl.BlockSpec((TB, D), lambda i: (i, 0)),
            pl.BlockSpec((TB, D), lambda i: (i, 0)),
        ),
        out_shape=(
            jax.ShapeDtypeStruct((T, D), _F32),
            jax.ShapeDtypeStruct((T, D), _BF),
        ),
        compiler_params=pltpu.CompilerParams(dimension_semantics=("arbitrary",)),
    )(gath, scale, apad, g, b)


def kernel(hidden_states, W_emb, b_emb, Wq, bq, Wk, bk, Wv, bv, Wo, bo,
           ln_attn_g, ln_attn_b, Wr, W1, b1, W2, b2, ln_out_g, ln_out_b):
    x = hidden_states.reshape(T, EMBD)
    r2 = lambda v: v.reshape(1, -1)
    b1h = b1.reshape(NE, 1, DFF)
    b2h = b2.reshape(NE, 1, D)
    h, hbf = _embed(x, W_emb, r2(b_emb))
    losses = []
    for _ in range(NLAYERS):
        ctx = _attn(hbf, Wq, r2(bq), Wk, r2(bk), Wv, r2(bv))
        a, code, code2, scale, loss = _router(
            ctx, Wo, r2(bo), h, r2(ln_attn_g), r2(ln_attn_b), Wr)
        sc_dispatch, sc_combine = _sc_kernels()
        buf = sc_dispatch(a, code.reshape(T))
        y = _ffn(buf, W1, b1h, W2, b2h)
        gath = sc_combine(y, code2.reshape(T))
        h, hbf = _lnout(gath, scale, a, r2(ln_out_g), r2(ln_out_b))
        losses.append(loss[0, 0])
    return h.reshape(1, T, D), jnp.stack(losses)
